# baseline (speedup 1.0000x reference)
import functools

import jax
import jax.numpy as jnp
from jax import lax
from jax.experimental import pallas as pl
from jax.experimental.pallas import tpu as pltpu
from jax.experimental.pallas import tpu_sc as plsc

V = 1_000_000
D = 64
B = 16384
L = 16
NC = 2
NS = 16
NW = NC * NS
NT = (V + 127) // 128
TPW = (NT + NW - 1) // NW
RD = 4
NB = 64
CH = 2048
LOCCAP = B + L
OUT_ROWS = B + NB
INVALID = 255

_mesh = plsc.VectorSubcoreMesh(core_axis_name="c", subcore_axis_name="s")

_IOTA = lambda: lax.broadcasted_iota(jnp.int32, (L,), 0)


def _splat(v):
    return jnp.zeros((L,), jnp.int32) + v


def _sget(ref, pos):
    base = pl.multiple_of((pos // L) * L, L)
    v = ref[pl.ds(base, L)]
    return jnp.max(jnp.where(_IOTA() == pos % L, v, -1))


@functools.partial(
    pl.kernel,
    out_type=jax.ShapeDtypeStruct((OUT_ROWS, 128), jnp.float32),
    mesh=_mesh,
    scratch_types=[
        pltpu.VMEM((2 * CH,), jnp.int32),
        pltpu.VMEM((LOCCAP,), jnp.int32),
        pltpu.VMEM((LOCCAP,), jnp.int32),
        pltpu.VMEM((256,), jnp.int32),
        pltpu.VMEM((256,), jnp.int32),
        pltpu.VMEM((256,), jnp.int32),
        pltpu.VMEM((256,), jnp.int32),
        pltpu.VMEM((L,), jnp.int32),
        pltpu.VMEM((L,), jnp.int32),
        pltpu.VMEM((RD, D, 128), jnp.float32),
        pltpu.VMEM((2, NB, 128), jnp.float32),
        pltpu.VMEM((L, NB), jnp.int32),
        pltpu.SemaphoreType.DMA,
        pltpu.SemaphoreType.DMA,
        pltpu.SemaphoreType.DMA,
        pltpu.SemaphoreType.DMA,
        pltpu.SemaphoreType.DMA,
        pltpu.SemaphoreType.DMA,
        pltpu.SemaphoreType.DMA,
        pltpu.SemaphoreType.DMA,
    ],
    compiler_params=pltpu.CompilerParams(needs_layout_passes=False),
)
def _gather_sc(xt_hbm, idx_hbm, interm_hbm, chunk_v, loc, sloc, hist, run0,
               cursor, tlist, sk1, sk2, ring, rowstage, ids_v, csem0, csem1,
               rsem0, rsem1, rsem2, rsem3, ssem0, ssem1):
    rsems = (rsem0, rsem1, rsem2, rsem3)
    csems = (csem0, csem1)
    ssems = (ssem0, ssem1)
    iota = _IOTA()
    wid = lax.axis_index("s") * NC + lax.axis_index("c")
    t_lo = wid * TPW
    t_hi = jnp.minimum(t_lo + TPW, NT)
    j_lo = t_lo * 128
    j_hi = t_hi * 128

    pltpu.async_copy(idx_hbm.at[pl.ds(0, CH)], chunk_v.at[pl.ds(0, CH)],
                     csems[0])

    def chunk_body(c, off_vec):
        nxt = c + 1

        @pl.when(nxt < B // CH)
        def _():
            off = pl.multiple_of(nxt * CH, L)
            for ph in range(2):
                @pl.when(nxt % 2 == ph)
                def _():
                    pltpu.async_copy(
                        idx_hbm.at[pl.ds(off, CH)],
                        chunk_v.at[pl.ds(ph * CH, CH)], csems[ph]
                    )

        def scan_chunk(ph):
            pltpu.make_async_copy(
                idx_hbm.at[pl.ds(0, CH)], chunk_v.at[pl.ds(ph * CH, CH)],
                csems[ph]
            ).wait()

            def scan_body(q, off_vec):
                base = pl.multiple_of(ph * CH + q * L, L)
                jv = chunk_v[pl.ds(base, L)]
                iv = c * CH + pl.multiple_of(q * L, L) + iota
                m = (jv >= j_lo) & (jv < j_hi)
                packed = (
                    (((jv >> 7) - t_lo) << 21) | ((jv & 127) << 14) | iv
                )
                slots = off_vec + plsc.cumsum(m.astype(jnp.int32)) - 1
                plsc.store_scatter(loc, [slots], packed, mask=m)
                return off_vec + plsc.all_reduce_population_count(m)

            return lax.fori_loop(0, CH // L, scan_body, off_vec)

        return lax.cond(c % 2 == 0, lambda: scan_chunk(0),
                        lambda: scan_chunk(1))

    off_vec = lax.fori_loop(0, B // CH, chunk_body, _splat(0))
    cnt = jnp.max(off_vec)
    plsc.store_scatter(loc, [off_vec + iota], _splat(INVALID << 21),
                       mask=iota >= 0)
    nq = (cnt + L - 1) // L

    def zero_body(k, _):
        base = pl.multiple_of(k * L, L)
        hist[pl.ds(base, L)] = _splat(0)
        return 0

    lax.fori_loop(0, 256 // L, zero_body, 0)

    def seg_sort(pv):
        ks, _ = plsc.sort_key_val(pv, iota)
        kt = ks >> 21
        sk1[pl.ds(0, L)] = kt
        prev = plsc.load_gather(sk1, [jnp.maximum(iota - 1, 0)])
        bnd = (iota == 0) | (kt != prev)
        sk2[pl.ds(0, L)] = bnd.astype(jnp.int32)
        nxtb = plsc.load_gather(sk2, [jnp.minimum(iota + 1, L - 1)])
        seg_start = plsc.cummax(jnp.where(bnd, iota, 0))
        rank = iota - seg_start
        seg_last = (iota == L - 1) | (nxtb > 0)
        return ks, kt, rank, seg_last

    def histA(q, _):
        base = pl.multiple_of(q * L, L)
        pv = loc[pl.ds(base, L)]
        ks, kt, rank, seg_last = seg_sort(pv)
        g = plsc.load_gather(hist, [kt])
        upd = seg_last & (kt < INVALID)
        plsc.store_scatter(hist, [kt], g + rank + 1, mask=upd)
        return 0

    lax.fori_loop(0, nq, histA, 0)

    def prefB(k, carry):
        base = pl.multiple_of(k * L, L)
        v = hist[pl.ds(base, L)]
        incl = plsc.cumsum(v)
        ex = carry + incl - v
        run0[pl.ds(base, L)] = ex
        cursor[pl.ds(base, L)] = ex
        return carry + jnp.max(incl)

    lax.fori_loop(0, 256 // L, prefB, 0)

    def placeC(q, _):
        base = pl.multiple_of(q * L, L)
        pv = loc[pl.ds(base, L)]
        ks, kt, rank, seg_last = seg_sort(pv)
        g = plsc.load_gather(cursor, [kt])
        ok = kt < INVALID
        plsc.store_scatter(sloc, [g + rank], ks, mask=ok)
        plsc.store_scatter(cursor, [kt], g + rank + 1, mask=seg_last & ok)
        return 0

    lax.fori_loop(0, nq, placeC, 0)

    def tl_body(k, f_vec):
        base = pl.multiple_of(k * L, L)
        fv = hist[pl.ds(base, L)]
        m = (fv > 0) & (base + iota < t_hi - t_lo)
        slots = f_vec + plsc.cumsum(m.astype(jnp.int32)) - 1
        plsc.store_scatter(tlist, [slots], base + iota, mask=m)
        return f_vec + plsc.all_reduce_population_count(m)

    f_vec = lax.fori_loop(0, 256 // L, tl_body, _splat(0))
    num_tiles = jnp.max(f_vec)

    def init_ids(row):
        for s in range(NB // L):
            plsc.store_scatter(
                ids_v,
                [_splat(row), pl.multiple_of(s * L, L) + iota],
                _splat(B + s * L) + iota,
                mask=iota >= 0,
            )

    init_ids(0)
    init_ids(8)

    def fire(b, p):
        tl = _sget(tlist, p)
        off = pl.multiple_of((t_lo + tl) * 128, 128)
        pltpu.async_copy(
            xt_hbm.at[:, pl.ds(off, 128)], ring.at[b], rsems[b]
        )

    for b in range(RD):
        @pl.when(b < num_tiles)
        def _():
            fire(b, b)

    def make_elem_body(b):
        def elem_body(e, carry):
            fill, b0, b1 = carry
            p = _sget(sloc, e)
            lane = (p >> 14) & 127
            im = p & ((1 << 14) - 1)
            rowslot = fill % NB
            ph = (fill // NB) % 2
            new_batch = rowslot == 0

            @pl.when(new_batch & (ph == 0) & (b0 >= 1))
            def _():
                pltpu.make_async_copy(
                    xt_hbm.at[:, pl.ds(0, 128)], rowstage.at[0], ssems[0]
                ).wait()

            @pl.when(new_batch & (ph == 1) & (b1 >= 1))
            def _():
                pltpu.make_async_copy(
                    xt_hbm.at[:, pl.ds(0, 128)], rowstage.at[1], ssems[1]
                ).wait()

            b0 = jnp.where(new_batch & (ph == 0) & (b0 >= 1), b0 - 1, b0)
            b1 = jnp.where(new_batch & (ph == 1) & (b1 >= 1), b1 - 1, b1)

            @pl.when(new_batch & (ph == 0))
            def _():
                init_ids(0)

            @pl.when(new_batch & (ph == 1))
            def _():
                init_ids(8)

            plsc.store_scatter(
                ids_v, [_splat(8 * ph), _splat(rowslot)], _splat(im),
                mask=iota == 0,
            )
            for q in range(D // L):
                vals = plsc.load_gather(
                    ring.at[b], [q * L + iota, _splat(lane)]
                )
                plsc.store_scatter(
                    rowstage,
                    [_splat(ph), _splat(rowslot), _splat(q * L) + iota],
                    vals,
                    mask=iota >= 0,
                )

            fill = fill + 1
            done = (fill % NB) == 0

            @pl.when(done & (ph == 0))
            def _():
                pltpu.async_copy(
                    rowstage.at[0], interm_hbm.at[ids_v.at[0]], ssems[0]
                )

            @pl.when(done & (ph == 1))
            def _():
                pltpu.async_copy(
                    rowstage.at[1], interm_hbm.at[ids_v.at[8]], ssems[1]
                )

            b0 = jnp.where(done & (ph == 0), b0 + 1, b0)
            b1 = jnp.where(done & (ph == 1), b1 + 1, b1)
            return fill, b0, b1

        return elem_body

    def round_body(r, carry):
        for b in range(RD):
            elem_body = make_elem_body(b)
            p = r * RD + b
            in_range = p < num_tiles

            @pl.when(in_range)
            def _():
                pltpu.make_async_copy(
                    xt_hbm.at[:, pl.ds(0, 128)], ring.at[b], rsems[b]
                ).wait()

            def run_tile(c):
                tl = _sget(tlist, p)
                s0 = _sget(run0, tl)
                e0 = s0 + _sget(hist, tl)
                return lax.fori_loop(s0, e0, elem_body, c)

            carry = lax.cond(in_range, run_tile, lambda c: c, carry)

            @pl.when((p + RD) < num_tiles)
            def _():
                fire(b, p + RD)
        return carry

    nr = (num_tiles + RD - 1) // RD
    fill, b0, b1 = lax.fori_loop(0, nr, round_body, (0, 0, 0))

    pending = (fill % NB) != 0
    phl = (fill // NB) % 2

    @pl.when(pending & (phl == 0))
    def _():
        pltpu.async_copy(
            rowstage.at[0], interm_hbm.at[ids_v.at[0]], ssems[0]
        )

    @pl.when(pending & (phl == 1))
    def _():
        pltpu.async_copy(
            rowstage.at[1], interm_hbm.at[ids_v.at[8]], ssems[1]
        )

    b0 = jnp.where(pending & (phl == 0), b0 + 1, b0)
    b1 = jnp.where(pending & (phl == 1), b1 + 1, b1)

    def drain0(i, _):
        pltpu.make_async_copy(
            xt_hbm.at[:, pl.ds(0, 128)], rowstage.at[0], ssems[0]
        ).wait()
        return 0

    def drain1(i, _):
        pltpu.make_async_copy(
            xt_hbm.at[:, pl.ds(0, 128)], rowstage.at[1], ssems[1]
        ).wait()
        return 0

    lax.fori_loop(0, b0, drain0, 0)
    lax.fori_loop(0, b1, drain1, 0)


BPW2 = B // 128 // NW


@functools.partial(
    pl.kernel,
    out_type=jax.ShapeDtypeStruct((D, B), jnp.float32),
    mesh=_mesh,
    scratch_types=[
        pltpu.VMEM((2, 128, 128), jnp.float32),
        pltpu.VMEM((2, D, 128), jnp.float32),
        pltpu.SemaphoreType.DMA,
        pltpu.SemaphoreType.DMA,
    ],
    compiler_params=pltpu.CompilerParams(needs_layout_passes=False),
)
def _finalize_sc(interm_hbm, out_hbm, src, dst, fsem0, fsem1):
    fsems = (fsem0, fsem1)
    iota = _IOTA()
    wid = lax.axis_index("s") * NC + lax.axis_index("c")
    blk0 = wid * BPW2

    pltpu.async_copy(
        interm_hbm.at[pl.ds(pl.multiple_of(blk0 * 128, 128), 128), :],
        src.at[0], fsems[0],
    )
    for k in range(BPW2):
        ph = k % 2
        nph = (k + 1) % 2
        if k + 1 < BPW2:
            pltpu.async_copy(
                interm_hbm.at[
                    pl.ds(pl.multiple_of((blk0 + k + 1) * 128, 128), 128), :
                ],
                src.at[nph], fsems[nph],
            )
        pltpu.make_async_copy(
            interm_hbm.at[pl.ds(0, 128), :], src.at[ph], fsems[ph]
        ).wait()

        def tr_body(c, _):
            for r8 in range(8):
                v = plsc.load_gather(
                    src.at[ph], [r8 * L + iota, _splat(c)]
                )
                plsc.store_scatter(
                    dst,
                    [_splat(ph), _splat(c), _splat(r8 * L) + iota],
                    v,
                    mask=iota >= 0,
                )
            return 0

        lax.fori_loop(0, D, tr_body, 0)
        pltpu.sync_copy(
            dst.at[ph],
            out_hbm.at[:, pl.ds(pl.multiple_of((blk0 + k) * 128, 128), 128)],
        )


def kernel(x, index):
    interm = _gather_sc(x.T, index.astype(jnp.int32))
    return _finalize_sc(interm).T


# speedup vs baseline: 1.1322x; 1.1322x over previous
import functools

import jax
import jax.numpy as jnp
from jax import lax
from jax.experimental import pallas as pl
from jax.experimental.pallas import tpu as pltpu
from jax.experimental.pallas import tpu_sc as plsc

V = 1_000_000
D = 64
B = 16384
L = 16
NC = 2
NS = 16
NW = NC * NS
NT = (V + 127) // 128
TPW = (NT + NW - 1) // NW
RD = 4
NB = 64
CH = 2048
LOCCAP = B + L
OUT_ROWS = B + NB
INVALID = 255

_mesh = plsc.VectorSubcoreMesh(core_axis_name="c", subcore_axis_name="s")

_IOTA = lambda: lax.broadcasted_iota(jnp.int32, (L,), 0)


def _splat(v):
    return jnp.zeros((L,), jnp.int32) + v


def _sget(ref, pos):
    base = pl.multiple_of((pos // L) * L, L)
    v = ref[pl.ds(base, L)]
    return jnp.max(jnp.where(_IOTA() == pos % L, v, -1))


@functools.partial(
    pl.kernel,
    out_type=jax.ShapeDtypeStruct((OUT_ROWS, 128), jnp.float32),
    mesh=_mesh,
    scratch_types=[
        pltpu.VMEM((2 * CH,), jnp.int32),
        pltpu.VMEM((LOCCAP,), jnp.int32),
        pltpu.VMEM((LOCCAP,), jnp.int32),
        pltpu.VMEM((256,), jnp.int32),
        pltpu.VMEM((256,), jnp.int32),
        pltpu.VMEM((256,), jnp.int32),
        pltpu.VMEM((256,), jnp.int32),
        pltpu.VMEM((L,), jnp.int32),
        pltpu.VMEM((L,), jnp.int32),
        pltpu.VMEM((RD, D, 128), jnp.float32),
        pltpu.VMEM((2, NB, 128), jnp.float32),
        pltpu.VMEM((L, NB), jnp.int32),
        pltpu.SemaphoreType.DMA,
        pltpu.SemaphoreType.DMA,
        pltpu.SemaphoreType.DMA,
        pltpu.SemaphoreType.DMA,
        pltpu.SemaphoreType.DMA,
        pltpu.SemaphoreType.DMA,
        pltpu.SemaphoreType.DMA,
        pltpu.SemaphoreType.DMA,
    ],
    compiler_params=pltpu.CompilerParams(needs_layout_passes=False),
)
def _gather_sc(xt_hbm, idx_hbm, interm_hbm, chunk_v, loc, sloc, hist, run0,
               cursor, tlist, sk1, sk2, ring, rowstage, ids_v, csem0, csem1,
               rsem0, rsem1, rsem2, rsem3, ssem0, ssem1):
    rsems = (rsem0, rsem1, rsem2, rsem3)
    csems = (csem0, csem1)
    ssems = (ssem0, ssem1)
    iota = _IOTA()
    wid = lax.axis_index("s") * NC + lax.axis_index("c")
    t_lo = wid * TPW
    t_hi = jnp.minimum(t_lo + TPW, NT)
    j_lo = t_lo * 128
    j_hi = t_hi * 128

    pltpu.async_copy(idx_hbm.at[pl.ds(0, CH)], chunk_v.at[pl.ds(0, CH)],
                     csems[0])

    def chunk_body(c, off_vec):
        nxt = c + 1

        @pl.when(nxt < B // CH)
        def _():
            off = pl.multiple_of(nxt * CH, L)
            for ph in range(2):
                @pl.when(nxt % 2 == ph)
                def _():
                    pltpu.async_copy(
                        idx_hbm.at[pl.ds(off, CH)],
                        chunk_v.at[pl.ds(ph * CH, CH)], csems[ph]
                    )

        def scan_chunk(ph):
            pltpu.make_async_copy(
                idx_hbm.at[pl.ds(0, CH)], chunk_v.at[pl.ds(ph * CH, CH)],
                csems[ph]
            ).wait()

            def scan_body(q, off_vec):
                for u in range(4):
                    base = pl.multiple_of(ph * CH + (q * 4 + u) * L, L)
                    jv = chunk_v[pl.ds(base, L)]
                    iv = (
                        c * CH + pl.multiple_of((q * 4 + u) * L, L) + iota
                    )
                    m = (jv >= j_lo) & (jv < j_hi)
                    packed = (
                        (((jv >> 7) - t_lo) << 21) | ((jv & 127) << 14) | iv
                    )
                    slots = off_vec + plsc.cumsum(m.astype(jnp.int32)) - 1
                    plsc.store_scatter(loc, [slots], packed, mask=m)
                    off_vec = off_vec + plsc.all_reduce_population_count(m)
                return off_vec

            return lax.fori_loop(0, CH // L // 4, scan_body, off_vec)

        return lax.cond(c % 2 == 0, lambda: scan_chunk(0),
                        lambda: scan_chunk(1))

    off_vec = lax.fori_loop(0, B // CH, chunk_body, _splat(0))
    cnt = jnp.max(off_vec)
    plsc.store_scatter(loc, [off_vec + iota], _splat(INVALID << 21),
                       mask=iota >= 0)
    nq = (cnt + L - 1) // L

    def zero_body(k, _):
        base = pl.multiple_of(k * L, L)
        hist[pl.ds(base, L)] = _splat(0)
        return 0

    lax.fori_loop(0, 256 // L, zero_body, 0)

    def seg_sort(pv):
        ks, _ = plsc.sort_key_val(pv, iota)
        kt = ks >> 21
        sk1[pl.ds(0, L)] = kt
        prev = plsc.load_gather(sk1, [jnp.maximum(iota - 1, 0)])
        bnd = (iota == 0) | (kt != prev)
        sk2[pl.ds(0, L)] = bnd.astype(jnp.int32)
        nxtb = plsc.load_gather(sk2, [jnp.minimum(iota + 1, L - 1)])
        seg_start = plsc.cummax(jnp.where(bnd, iota, 0))
        rank = iota - seg_start
        seg_last = (iota == L - 1) | (nxtb > 0)
        return ks, kt, rank, seg_last

    def histA(q, _):
        base = pl.multiple_of(q * L, L)
        pv = loc[pl.ds(base, L)]
        ks, kt, rank, seg_last = seg_sort(pv)
        g = plsc.load_gather(hist, [kt])
        upd = seg_last & (kt < INVALID)
        plsc.store_scatter(hist, [kt], g + rank + 1, mask=upd)
        return 0

    lax.fori_loop(0, nq, histA, 0)

    def prefB(k, carry):
        base = pl.multiple_of(k * L, L)
        v = hist[pl.ds(base, L)]
        incl = plsc.cumsum(v)
        ex = carry + incl - v
        run0[pl.ds(base, L)] = ex
        cursor[pl.ds(base, L)] = ex
        return carry + jnp.max(incl)

    lax.fori_loop(0, 256 // L, prefB, 0)

    def placeC(q, _):
        base = pl.multiple_of(q * L, L)
        pv = loc[pl.ds(base, L)]
        ks, kt, rank, seg_last = seg_sort(pv)
        g = plsc.load_gather(cursor, [kt])
        ok = kt < INVALID
        plsc.store_scatter(sloc, [g + rank], ks, mask=ok)
        plsc.store_scatter(cursor, [kt], g + rank + 1, mask=seg_last & ok)
        return 0

    lax.fori_loop(0, nq, placeC, 0)

    def tl_body(k, f_vec):
        base = pl.multiple_of(k * L, L)
        fv = hist[pl.ds(base, L)]
        m = (fv > 0) & (base + iota < t_hi - t_lo)
        slots = f_vec + plsc.cumsum(m.astype(jnp.int32)) - 1
        plsc.store_scatter(tlist, [slots], base + iota, mask=m)
        return f_vec + plsc.all_reduce_population_count(m)

    f_vec = lax.fori_loop(0, 256 // L, tl_body, _splat(0))
    num_tiles = jnp.max(f_vec)

    def init_ids(row):
        for s in range(NB // L):
            plsc.store_scatter(
                ids_v,
                [_splat(row), pl.multiple_of(s * L, L) + iota],
                _splat(B + s * L) + iota,
                mask=iota >= 0,
            )

    init_ids(0)
    init_ids(8)

    def fire(b, p):
        tl = _sget(tlist, p)
        off = pl.multiple_of((t_lo + tl) * 128, 128)
        pltpu.async_copy(
            xt_hbm.at[:, pl.ds(off, 128)], ring.at[b], rsems[b]
        )

    for b in range(RD):
        @pl.when(b < num_tiles)
        def _():
            fire(b, b)

    def make_elem_body(b):
        def elem_body(e, carry):
            fill, b0, b1 = carry
            p = _sget(sloc, e)
            lane = (p >> 14) & 127
            im = p & ((1 << 14) - 1)
            rowslot = fill % NB
            ph = (fill // NB) % 2
            new_batch = rowslot == 0

            @pl.when(new_batch & (ph == 0) & (b0 >= 1))
            def _():
                pltpu.make_async_copy(
                    xt_hbm.at[:, pl.ds(0, 128)], rowstage.at[0], ssems[0]
                ).wait()

            @pl.when(new_batch & (ph == 1) & (b1 >= 1))
            def _():
                pltpu.make_async_copy(
                    xt_hbm.at[:, pl.ds(0, 128)], rowstage.at[1], ssems[1]
                ).wait()

            b0 = jnp.where(new_batch & (ph == 0) & (b0 >= 1), b0 - 1, b0)
            b1 = jnp.where(new_batch & (ph == 1) & (b1 >= 1), b1 - 1, b1)

            @pl.when(new_batch & (ph == 0))
            def _():
                init_ids(0)

            @pl.when(new_batch & (ph == 1))
            def _():
                init_ids(8)

            plsc.store_scatter(
                ids_v, [_splat(8 * ph), _splat(rowslot)], _splat(im),
                mask=iota == 0,
            )
            for q in range(D // L):
                vals = plsc.load_gather(
                    ring.at[b], [q * L + iota, _splat(lane)]
                )
                plsc.store_scatter(
                    rowstage,
                    [_splat(ph), _splat(rowslot), _splat(q * L) + iota],
                    vals,
                    mask=iota >= 0,
                )

            fill = fill + 1
            done = (fill % NB) == 0

            @pl.when(done & (ph == 0))
            def _():
                pltpu.async_copy(
                    rowstage.at[0], interm_hbm.at[ids_v.at[0]], ssems[0]
                )

            @pl.when(done & (ph == 1))
            def _():
                pltpu.async_copy(
                    rowstage.at[1], interm_hbm.at[ids_v.at[8]], ssems[1]
                )

            b0 = jnp.where(done & (ph == 0), b0 + 1, b0)
            b1 = jnp.where(done & (ph == 1), b1 + 1, b1)
            return fill, b0, b1

        return elem_body

    def round_body(r, carry):
        for b in range(RD):
            elem_body = make_elem_body(b)
            p = r * RD + b
            in_range = p < num_tiles

            @pl.when(in_range)
            def _():
                pltpu.make_async_copy(
                    xt_hbm.at[:, pl.ds(0, 128)], ring.at[b], rsems[b]
                ).wait()

            def run_tile(c):
                tl = _sget(tlist, p)
                s0 = _sget(run0, tl)
                e0 = s0 + _sget(hist, tl)
                return lax.fori_loop(s0, e0, elem_body, c)

            carry = lax.cond(in_range, run_tile, lambda c: c, carry)

            @pl.when((p + RD) < num_tiles)
            def _():
                fire(b, p + RD)
        return carry

    nr = (num_tiles + RD - 1) // RD
    fill, b0, b1 = lax.fori_loop(0, nr, round_body, (0, 0, 0))

    pending = (fill % NB) != 0
    phl = (fill // NB) % 2

    @pl.when(pending & (phl == 0))
    def _():
        pltpu.async_copy(
            rowstage.at[0], interm_hbm.at[ids_v.at[0]], ssems[0]
        )

    @pl.when(pending & (phl == 1))
    def _():
        pltpu.async_copy(
            rowstage.at[1], interm_hbm.at[ids_v.at[8]], ssems[1]
        )

    b0 = jnp.where(pending & (phl == 0), b0 + 1, b0)
    b1 = jnp.where(pending & (phl == 1), b1 + 1, b1)

    def drain0(i, _):
        pltpu.make_async_copy(
            xt_hbm.at[:, pl.ds(0, 128)], rowstage.at[0], ssems[0]
        ).wait()
        return 0

    def drain1(i, _):
        pltpu.make_async_copy(
            xt_hbm.at[:, pl.ds(0, 128)], rowstage.at[1], ssems[1]
        ).wait()
        return 0

    lax.fori_loop(0, b0, drain0, 0)
    lax.fori_loop(0, b1, drain1, 0)


def kernel(x, index):
    interm = _gather_sc(x.T, index.astype(jnp.int32))
    return interm[:B, :D]
